# baseline (speedup 1.0000x reference)
import jax
import jax.numpy as jnp
from jax import lax
from jax.experimental import pallas as pl
from jax.experimental.pallas import tpu as pltpu
from jax.experimental.pallas import tpu_sc as plsc

_K = 512
_D = 64
_N = 32768
_N2 = _N // 2
_T2 = 1024
_CK = 128
_NB = _N2 // _T2


def _tc_body(x2_ref, emb_ref, idxe_ref, idxo_ref):
    iota_f = lax.broadcasted_iota(jnp.int32, (_T2, _CK), 1).astype(jnp.float32)
    for out_ref, lo in ((idxe_ref, 0), (idxo_ref, _D)):
        x = x2_ref[:, lo:lo + _D]
        q2 = jnp.sum(x * x, axis=1, keepdims=True)
        m_lane = jnp.zeros((_T2, _CK), jnp.float32)
        c_lane = jnp.zeros((_T2, _CK), jnp.float32)
        for c in range(_K // _CK):
            embc = emb_ref[pl.ds(c * _CK, _CK), :]
            dotc = lax.dot_general(
                x, embc, (((1,), (1,)), ((), ())),
                preferred_element_type=jnp.float32)
            e2c = jnp.sum(embc * embc, axis=1)
            d2c = (q2 + e2c[None, :]) - 2.0 * dotc
            if c == 0:
                m_lane = d2c
            else:
                upd = d2c < m_lane
                m_lane = jnp.where(upd, d2c, m_lane)
                c_lane = jnp.where(upd, c * _CK * 1.0, c_lane)
        m = jnp.min(m_lane, axis=1, keepdims=True)
        kcand = jnp.where(m_lane == m, c_lane + iota_f, 1e9)
        k = jnp.min(kcand, axis=1)
        out_ref[...] = k.astype(jnp.int32).reshape(1, 8, _T2 // 8)


def _tc_closest(x2, embed):
    return pl.pallas_call(
        _tc_body,
        grid=(_NB,),
        in_specs=[
            pl.BlockSpec((_T2, 2 * _D), lambda i: (i, 0)),
            pl.BlockSpec((_K, _D), lambda i: (0, 0)),
        ],
        out_specs=[
            pl.BlockSpec((1, 8, _T2 // 8), lambda i: (i, 0, 0)),
            pl.BlockSpec((1, 8, _T2 // 8), lambda i: (i, 0, 0)),
        ],
        out_shape=[
            jax.ShapeDtypeStruct((_NB, 8, _T2 // 8), jnp.int32),
            jax.ShapeDtypeStruct((_NB, 8, _T2 // 8), jnp.int32),
        ],
    )(x2, embed)


_NC = 2
_NS = 16
_NW = _NC * _NS
_RW = _N2 // _NW
_S2 = 128
_NCH = _RW // _S2


def _sc_body(idxe_hbm, idxo_hbm, x2_hbm, emb_hbm, out_hbm, part_hbm,
             ie_v, io_v, re_v, ro_v, x_v, acc_v, sems):
    wid = lax.axis_index("s") * _NC + lax.axis_index("c")
    r0w = wid * _RW

    def start(j, slot):
        r0 = r0w + j * _S2
        blk = r0 // _S2
        pltpu.sync_copy(idxe_hbm.at[blk // 8, blk % 8], ie_v.at[slot])
        pltpu.sync_copy(idxo_hbm.at[blk // 8, blk % 8], io_v.at[slot])
        h1 = pltpu.async_copy(emb_hbm.at[ie_v.at[slot]], re_v.at[slot], sems.at[slot])
        h2 = pltpu.async_copy(emb_hbm.at[io_v.at[slot]], ro_v.at[slot], sems.at[slot])
        h3 = pltpu.async_copy(x2_hbm.at[pl.ds(r0, _S2), :], x_v.at[slot], sems.at[slot])
        return (h1, h2, h3)

    pending = start(0, 0)
    acc = jnp.zeros((16,), jnp.float32)
    for j in range(_NCH):
        slot = j % 2
        for h in pending:
            h.wait()
        if j + 1 < _NCH:
            pending = start(j + 1, (j + 1) % 2)

        def row(i, acc):
            for k in range(_D // 16):
                sle = pl.ds(k * 16, 16)
                slo = pl.ds(_D + k * 16, 16)
                xe = x_v[slot, i, sle]
                te = re_v[slot, i, sle] - xe
                x_v[slot, i, sle] = xe + te
                acc = acc + te * te
                xo = x_v[slot, i, slo]
                to = ro_v[slot, i, sle] - xo
                x_v[slot, i, slo] = xo + to
                acc = acc + to * to
            return acc


        acc = lax.fori_loop(0, _S2, row, acc)
        pltpu.sync_copy(x_v.at[slot], out_hbm.at[pl.ds(r0w + j * _S2, _S2), :])
    acc_v[...] = acc
    pltpu.sync_copy(acc_v, part_hbm.at[wid])


def _sc_gather(idx_e, idx_o, x2, embed):
    mesh = plsc.VectorSubcoreMesh(core_axis_name="c", subcore_axis_name="s")
    f = pl.kernel(
        _sc_body,
        mesh=mesh,
        out_type=[
            jax.ShapeDtypeStruct((_N2, 2 * _D), jnp.float32),
            jax.ShapeDtypeStruct((_NW, 16), jnp.float32),
        ],
        scratch_types=[
            pltpu.VMEM((2, _S2), jnp.int32),
            pltpu.VMEM((2, _S2), jnp.int32),
            pltpu.VMEM((2, _S2, _D), jnp.float32),
            pltpu.VMEM((2, _S2, _D), jnp.float32),
            pltpu.VMEM((2, _S2, 2 * _D), jnp.float32),
            pltpu.VMEM((16,), jnp.float32),
            pltpu.SemaphoreType.DMA((2,)),
        ],
        compiler_params=pltpu.CompilerParams(use_tc_tiling_on_sc=False),
    )
    return f(idx_e, idx_o, x2, embed)


def kernel(enc, embed):
    B, C, H, W = enc.shape
    x2 = enc.reshape(_N2, 2 * _D)
    idx_e, idx_o = _tc_closest(x2, embed)
    out2, partials = _sc_gather(idx_e, idx_o, x2, embed)
    mse = jnp.sum(partials) / jnp.float32(_N * _D)
    quantize_loss = mse + mse
    closest = jnp.stack(
        [idx_e.reshape(_N2), idx_o.reshape(_N2)], axis=1).reshape(B, _N // B)
    return (out2.reshape(B, C, H, W), quantize_loss, closest)


# speedup vs baseline: 1.0094x; 1.0094x over previous
import jax
import jax.numpy as jnp
from jax import lax
from jax.experimental import pallas as pl
from jax.experimental.pallas import tpu as pltpu
from jax.experimental.pallas import tpu_sc as plsc

_K = 512
_D = 64
_N = 32768
_N2 = _N // 2
_T2 = 1024
_CK = 128
_NB = _N2 // _T2


def _tc_body(x2_ref, emb_ref, idxe_ref, idxo_ref):
    iota_f = lax.broadcasted_iota(jnp.int32, (_T2, _CK), 1).astype(jnp.float32)
    for out_ref, lo in ((idxe_ref, 0), (idxo_ref, _D)):
        x = x2_ref[:, lo:lo + _D]
        q2 = jnp.sum(x * x, axis=1, keepdims=True)
        m_lane = jnp.zeros((_T2, _CK), jnp.float32)
        c_lane = jnp.zeros((_T2, _CK), jnp.float32)
        for c in range(_K // _CK):
            embc = emb_ref[pl.ds(c * _CK, _CK), :]
            dotc = lax.dot_general(
                x, embc, (((1,), (1,)), ((), ())),
                preferred_element_type=jnp.float32)
            e2c = jnp.sum(embc * embc, axis=1)
            d2c = (q2 + e2c[None, :]) - 2.0 * dotc
            if c == 0:
                m_lane = d2c
            else:
                upd = d2c < m_lane
                m_lane = jnp.where(upd, d2c, m_lane)
                c_lane = jnp.where(upd, c * _CK * 1.0, c_lane)
        m = jnp.min(m_lane, axis=1, keepdims=True)
        kcand = jnp.where(m_lane == m, c_lane + iota_f, 1e9)
        k = jnp.min(kcand, axis=1)
        out_ref[...] = k.astype(jnp.int32).reshape(1, 8, _T2 // 8)


def _tc_closest(x2, embed):
    return pl.pallas_call(
        _tc_body,
        grid=(_NB,),
        in_specs=[
            pl.BlockSpec((_T2, 2 * _D), lambda i: (i, 0)),
            pl.BlockSpec((_K, _D), lambda i: (0, 0)),
        ],
        out_specs=[
            pl.BlockSpec((1, 8, _T2 // 8), lambda i: (i, 0, 0)),
            pl.BlockSpec((1, 8, _T2 // 8), lambda i: (i, 0, 0)),
        ],
        out_shape=[
            jax.ShapeDtypeStruct((_NB, 8, _T2 // 8), jnp.int32),
            jax.ShapeDtypeStruct((_NB, 8, _T2 // 8), jnp.int32),
        ],
    )(x2, embed)


_NC = 2
_NS = 16
_NW = _NC * _NS
_RW = _N2 // _NW
_S2 = 128
_NCH = _RW // _S2


def _sc_body(idxe_hbm, idxo_hbm, x2_hbm, emb_hbm, out_hbm, part_hbm,
             ie_v, io_v, re_v, ro_v, x_v, acc_v, sems):
    wid = lax.axis_index("s") * _NC + lax.axis_index("c")
    r0w = wid * _RW

    def start(j, slot):
        r0 = r0w + j * _S2
        blk = r0 // _S2
        pltpu.sync_copy(idxe_hbm.at[blk // 8, blk % 8], ie_v.at[slot])
        pltpu.sync_copy(idxo_hbm.at[blk // 8, blk % 8], io_v.at[slot])
        h1 = pltpu.async_copy(emb_hbm.at[ie_v.at[slot]], re_v.at[slot], sems.at[slot])
        h2 = pltpu.async_copy(emb_hbm.at[io_v.at[slot]], ro_v.at[slot], sems.at[slot])
        h3 = pltpu.async_copy(x2_hbm.at[pl.ds(r0, _S2), :], x_v.at[slot], sems.at[slot])
        return (h1, h2, h3)

    pending = start(0, 0)
    acc = jnp.zeros((16,), jnp.float32)
    for j in range(_NCH):
        slot = j % 2
        for h in pending:
            h.wait()
        if j + 1 < _NCH:
            pending = start(j + 1, (j + 1) % 2)

        def row(i, acc):
            for k in range(_D // 16):
                sle = pl.ds(k * 16, 16)
                slo = pl.ds(_D + k * 16, 16)
                xe = x_v[slot, i, sle]
                te = re_v[slot, i, sle] - xe
                x_v[slot, i, sle] = xe + te
                acc = acc + te * te
                xo = x_v[slot, i, slo]
                to = ro_v[slot, i, sle] - xo
                x_v[slot, i, slo] = xo + to
                acc = acc + to * to
            return acc


        acc = lax.fori_loop(0, _S2, row, acc)
        pltpu.sync_copy(x_v.at[slot], out_hbm.at[pl.ds(r0w + j * _S2, _S2), :])
    acc_v[...] = acc
    pltpu.sync_copy(acc_v, part_hbm.at[wid])


def _sc_gather(idx_e, idx_o, x2, embed):
    embp = jnp.concatenate([embed, jnp.zeros((_K, _D), jnp.float32)], axis=1)
    mesh = plsc.VectorSubcoreMesh(core_axis_name="c", subcore_axis_name="s")
    f = pl.kernel(
        _sc_body,
        mesh=mesh,
        out_type=[
            jax.ShapeDtypeStruct((_N2, 2 * _D), jnp.float32),
            jax.ShapeDtypeStruct((_NW, 16), jnp.float32),
        ],
        scratch_types=[
            pltpu.VMEM((2, _S2), jnp.int32),
            pltpu.VMEM((2, _S2), jnp.int32),
            pltpu.VMEM((2, _S2, 2 * _D), jnp.float32),
            pltpu.VMEM((2, _S2, 2 * _D), jnp.float32),
            pltpu.VMEM((2, _S2, 2 * _D), jnp.float32),
            pltpu.VMEM((16,), jnp.float32),
            pltpu.SemaphoreType.DMA((2,)),
        ],
    )
    return f(idx_e, idx_o, x2, embp)


def kernel(enc, embed):
    B, C, H, W = enc.shape
    x2 = enc.reshape(_N2, 2 * _D)
    idx_e, idx_o = _tc_closest(x2, embed)
    out2, partials = _sc_gather(idx_e, idx_o, x2, embed)
    mse = jnp.sum(partials) / jnp.float32(_N * _D)
    quantize_loss = mse + mse
    closest = jnp.stack(
        [idx_e.reshape(_N2), idx_o.reshape(_N2)], axis=1).reshape(B, _N // B)
    return (out2.reshape(B, C, H, W), quantize_loss, closest)
